# baseline (device time: 34879 ns/iter reference)
import jax
import jax.numpy as jnp
from jax import lax
from jax.experimental import pallas as pl
from jax.experimental.pallas import tpu as pltpu

N_DEV = 4
N_LAYERS = 3


def kernel(x, Win0, Wout0, Win1, Wout1, Win2, Wout2):
    b, d = x.shape
    hsh = Win0.shape[1]

    def body(x_ref, win0, wout0, win1, wout1, win2, wout2, out_ref,
             sin, sout, winbf, woutbf, comm,
             dma_sems, send_sems, recv_sems):
        my = lax.axis_index("i")
        left = lax.rem(my + N_DEV - 1, N_DEV)
        right = lax.rem(my + 1, N_DEV)

        wins_hbm = (win0, win1, win2)
        wouts_hbm = (wout0, wout1, wout2)

        def dma_in(w_hbm, stage_ref, slot, sem_idx):
            return pltpu.make_async_copy(
                w_hbm, stage_ref.at[slot], dma_sems.at[sem_idx])

        dma_in(wins_hbm[0], sin, 0, 0).start()
        dma_in(wouts_hbm[0], sout, 0, 1).start()
        dma_in(wins_hbm[1], sin, 1, 2).start()
        dma_in(wouts_hbm[1], sout, 1, 3).start()

        barrier = pltpu.get_barrier_semaphore()
        for nbr in (left, right):
            pl.semaphore_signal(barrier, inc=1, device_id=(nbr,),
                                device_id_type=pl.DeviceIdType.MESH)
        pl.semaphore_wait(barrier, 2)

        def cast_win(li):
            slot = li % 2
            dma_in(wins_hbm[li], sin, slot, 2 * li).wait()
            winbf[slot] = sin[slot].astype(jnp.bfloat16)

        def cast_wout(li):
            slot = li % 2
            dma_in(wouts_hbm[li], sout, slot, 2 * li + 1).wait()
            woutbf[slot] = sout[slot].astype(jnp.bfloat16)

        cast_win(0)

        x_val = x_ref[...]
        for li in range(N_LAYERS):
            slot = li % 2
            h = jnp.dot(x_val.astype(jnp.bfloat16), winbf[slot],
                        preferred_element_type=jnp.float32)
            h = jnp.maximum(h, 0.0)
            if li == 0:
                cast_wout(0)
            partial = jnp.dot(h.astype(jnp.bfloat16), woutbf[slot],
                              preferred_element_type=jnp.float32)

            s = 3 * li
            comm[li, 0] = partial.astype(jnp.bfloat16)
            r1a = pltpu.make_async_remote_copy(
                src_ref=comm.at[li, 0], dst_ref=comm.at[li, 1],
                send_sem=send_sems.at[s], recv_sem=recv_sems.at[s],
                device_id=(right,), device_id_type=pl.DeviceIdType.MESH)
            r1b = pltpu.make_async_remote_copy(
                src_ref=comm.at[li, 0], dst_ref=comm.at[li, 2],
                send_sem=send_sems.at[s + 1], recv_sem=recv_sems.at[s + 1],
                device_id=(left,), device_id_type=pl.DeviceIdType.MESH)
            r1a.start()
            r1b.start()

            if li + 1 < N_LAYERS:
                cast_win(li + 1)
                if li + 2 < N_LAYERS:
                    dma_in(wins_hbm[li + 2], sin, li % 2, 2 * (li + 2)).start()

            r1a.wait_recv()
            r2 = pltpu.make_async_remote_copy(
                src_ref=comm.at[li, 1], dst_ref=comm.at[li, 3],
                send_sem=send_sems.at[s + 2], recv_sem=recv_sems.at[s + 2],
                device_id=(right,), device_id_type=pl.DeviceIdType.MESH)
            r2.start()

            if li + 1 < N_LAYERS:
                cast_wout(li + 1)
                if li + 2 < N_LAYERS:
                    dma_in(wouts_hbm[li + 2], sout, li % 2,
                           2 * (li + 2) + 1).start()

            r1b.wait_recv()
            acc = (partial
                   + comm[li, 1].astype(jnp.float32)
                   + comm[li, 2].astype(jnp.float32))
            r2.wait_recv()
            x_val = acc + comm[li, 3].astype(jnp.float32)

            r1a.wait_send()
            r1b.wait_send()
            r2.wait_send()

        out_ref[...] = x_val

    return pl.pallas_call(
        body,
        out_shape=jax.ShapeDtypeStruct((b, d), jnp.float32),
        in_specs=[pl.BlockSpec(memory_space=pltpu.VMEM)]
        + [pl.BlockSpec(memory_space=pltpu.MemorySpace.HBM)] * 6,
        out_specs=pl.BlockSpec(memory_space=pltpu.VMEM),
        scratch_shapes=[
            pltpu.VMEM((2, d, hsh), jnp.float32),
            pltpu.VMEM((2, hsh, d), jnp.float32),
            pltpu.VMEM((2, d, hsh), jnp.bfloat16),
            pltpu.VMEM((2, hsh, d), jnp.bfloat16),
            pltpu.VMEM((N_LAYERS, 4, b, d), jnp.bfloat16),
            pltpu.SemaphoreType.DMA((2 * N_LAYERS,)),
            pltpu.SemaphoreType.DMA((3 * N_LAYERS,)),
            pltpu.SemaphoreType.DMA((3 * N_LAYERS,)),
        ],
        compiler_params=pltpu.CompilerParams(
            collective_id=0,
            vmem_limit_bytes=100 * 1024 * 1024,
        ),
    )(x, Win0, Wout0, Win1, Wout1, Win2, Wout2)


# device time: 31319 ns/iter; 1.1137x vs baseline; 1.1137x over previous
import jax
import jax.numpy as jnp
from jax import lax
from jax.experimental import pallas as pl
from jax.experimental.pallas import tpu as pltpu

N_DEV = 4
N_LAYERS = 3


def kernel(x, Win0, Wout0, Win1, Wout1, Win2, Wout2):
    b, d = x.shape
    hsh = Win0.shape[1]

    def body(x_ref, win0, wout0, win1, wout1, win2, wout2, out_ref,
             sin, sout, winbf, woutbf, comm,
             dma_sems, send_sems, recv_sems):
        my = lax.axis_index("i")
        left = lax.rem(my + N_DEV - 1, N_DEV)
        right = lax.rem(my + 1, N_DEV)

        wins_hbm = (win0, win1, win2)
        wouts_hbm = (wout0, wout1, wout2)

        def dma_in(w_hbm, stage_ref, slot, sem_idx):
            return pltpu.make_async_copy(
                w_hbm, stage_ref.at[slot], dma_sems.at[sem_idx])

        dma_in(wins_hbm[0], sin, 0, 0).start()
        dma_in(wouts_hbm[0], sout, 0, 1).start()
        dma_in(wins_hbm[1], sin, 1, 2).start()
        dma_in(wouts_hbm[1], sout, 1, 3).start()

        barrier = pltpu.get_barrier_semaphore()
        for nbr in (left, right, lax.rem(my + 2, N_DEV)):
            pl.semaphore_signal(barrier, inc=1, device_id=(nbr,),
                                device_id_type=pl.DeviceIdType.MESH)
        pl.semaphore_wait(barrier, 3)

        def cast_win(li):
            slot = li % 2
            dma_in(wins_hbm[li], sin, slot, 2 * li).wait()
            winbf[slot] = sin[slot].astype(jnp.bfloat16)

        def cast_wout(li):
            slot = li % 2
            dma_in(wouts_hbm[li], sout, slot, 2 * li + 1).wait()
            woutbf[slot] = sout[slot].astype(jnp.bfloat16)

        cast_win(0)

        x_val = x_ref[...]
        for li in range(N_LAYERS):
            slot = li % 2
            h = jnp.dot(x_val.astype(jnp.bfloat16), winbf[slot],
                        preferred_element_type=jnp.float32)
            h = jnp.maximum(h, 0.0)
            if li == 0:
                cast_wout(0)
            partial = jnp.dot(h.astype(jnp.bfloat16), woutbf[slot],
                              preferred_element_type=jnp.float32)

            s = 3 * li
            diag = lax.rem(my + 2, N_DEV)
            comm[li, 0] = partial.astype(jnp.bfloat16)
            r_r = pltpu.make_async_remote_copy(
                src_ref=comm.at[li, 0], dst_ref=comm.at[li, 1],
                send_sem=send_sems.at[s], recv_sem=recv_sems.at[s],
                device_id=(right,), device_id_type=pl.DeviceIdType.MESH)
            r_l = pltpu.make_async_remote_copy(
                src_ref=comm.at[li, 0], dst_ref=comm.at[li, 2],
                send_sem=send_sems.at[s + 1], recv_sem=recv_sems.at[s + 1],
                device_id=(left,), device_id_type=pl.DeviceIdType.MESH)
            r_d = pltpu.make_async_remote_copy(
                src_ref=comm.at[li, 0], dst_ref=comm.at[li, 3],
                send_sem=send_sems.at[s + 2], recv_sem=recv_sems.at[s + 2],
                device_id=(diag,), device_id_type=pl.DeviceIdType.MESH)
            r_r.start()
            r_l.start()
            r_d.start()

            if li + 1 < N_LAYERS:
                cast_win(li + 1)
                if li + 2 < N_LAYERS:
                    dma_in(wins_hbm[li + 2], sin, li % 2, 2 * (li + 2)).start()
                cast_wout(li + 1)
                if li + 2 < N_LAYERS:
                    dma_in(wouts_hbm[li + 2], sout, li % 2,
                           2 * (li + 2) + 1).start()

            r_r.wait_recv()
            r_l.wait_recv()
            acc = (partial
                   + comm[li, 1].astype(jnp.float32)
                   + comm[li, 2].astype(jnp.float32))
            r_d.wait_recv()
            x_val = acc + comm[li, 3].astype(jnp.float32)

            r_r.wait_send()
            r_l.wait_send()
            r_d.wait_send()

        out_ref[...] = x_val

    return pl.pallas_call(
        body,
        out_shape=jax.ShapeDtypeStruct((b, d), jnp.float32),
        in_specs=[pl.BlockSpec(memory_space=pltpu.VMEM)]
        + [pl.BlockSpec(memory_space=pltpu.MemorySpace.HBM)] * 6,
        out_specs=pl.BlockSpec(memory_space=pltpu.VMEM),
        scratch_shapes=[
            pltpu.VMEM((2, d, hsh), jnp.float32),
            pltpu.VMEM((2, hsh, d), jnp.float32),
            pltpu.VMEM((2, d, hsh), jnp.bfloat16),
            pltpu.VMEM((2, hsh, d), jnp.bfloat16),
            pltpu.VMEM((N_LAYERS, 4, b, d), jnp.bfloat16),
            pltpu.SemaphoreType.DMA((2 * N_LAYERS,)),
            pltpu.SemaphoreType.DMA((3 * N_LAYERS,)),
            pltpu.SemaphoreType.DMA((3 * N_LAYERS,)),
        ],
        compiler_params=pltpu.CompilerParams(
            collective_id=0,
            vmem_limit_bytes=100 * 1024 * 1024,
        ),
    )(x, Win0, Wout0, Win1, Wout1, Win2, Wout2)


# device time: 31317 ns/iter; 1.1137x vs baseline; 1.0001x over previous
import os

import jax
import jax.numpy as jnp
from jax import lax
from jax.experimental import pallas as pl
from jax.experimental.pallas import tpu as pltpu

_WDTYPE = os.environ.get("WDTYPE", "bf16")

N_DEV = 4
N_LAYERS = 3


def kernel(x, Win0, Wout0, Win1, Wout1, Win2, Wout2):
    b, d = x.shape
    hsh = Win0.shape[1]

    def body(x_ref, win0, wout0, win1, wout1, win2, wout2, out_ref,
             sin, sout, winbf, woutbf, comm,
             dma_sems, send_sems, recv_sems):
        my = lax.axis_index("i")
        left = lax.rem(my + N_DEV - 1, N_DEV)
        right = lax.rem(my + 1, N_DEV)

        wins_hbm = (win0, win1, win2)
        wouts_hbm = (wout0, wout1, wout2)

        def dma_in(w_hbm, stage_ref, slot, sem_idx):
            return pltpu.make_async_copy(
                w_hbm, stage_ref.at[slot], dma_sems.at[sem_idx])

        dma_in(wins_hbm[0], sin, 0, 0).start()
        dma_in(wouts_hbm[0], sout, 0, 1).start()
        dma_in(wins_hbm[1], sin, 1, 2).start()
        dma_in(wouts_hbm[1], sout, 1, 3).start()

        barrier = pltpu.get_barrier_semaphore()
        for nbr in (left, right, lax.rem(my + 2, N_DEV)):
            pl.semaphore_signal(barrier, inc=1, device_id=(nbr,),
                                device_id_type=pl.DeviceIdType.MESH)
        pl.semaphore_wait(barrier, 3)

        def cast_win(li):
            slot = li % 2
            dma_in(wins_hbm[li], sin, slot, 2 * li).wait()
            if _WDTYPE == "bf16":
                winbf[slot] = sin[slot].astype(jnp.bfloat16)

        def cast_wout(li):
            slot = li % 2
            dma_in(wouts_hbm[li], sout, slot, 2 * li + 1).wait()
            if _WDTYPE == "bf16":
                woutbf[slot] = sout[slot].astype(jnp.bfloat16)

        cast_win(0)

        x_val = x_ref[...]
        for li in range(N_LAYERS):
            slot = li % 2
            if _WDTYPE == "bf16":
                h = jnp.dot(x_val.astype(jnp.bfloat16), winbf[slot],
                            preferred_element_type=jnp.float32)
            else:
                h = jnp.dot(x_val, sin[slot],
                            preferred_element_type=jnp.float32)
            h = jnp.maximum(h, 0.0)
            if li == 0:
                cast_wout(0)
            if _WDTYPE == "bf16":
                partial = jnp.dot(h.astype(jnp.bfloat16), woutbf[slot],
                                  preferred_element_type=jnp.float32)
            else:
                partial = jnp.dot(h, sout[slot],
                                  preferred_element_type=jnp.float32)

            s = 3 * li
            diag = lax.rem(my + 2, N_DEV)
            comm[li, 0] = partial.astype(jnp.bfloat16)
            r_r = pltpu.make_async_remote_copy(
                src_ref=comm.at[li, 0], dst_ref=comm.at[li, 1],
                send_sem=send_sems.at[s], recv_sem=recv_sems.at[s],
                device_id=(right,), device_id_type=pl.DeviceIdType.MESH)
            r_l = pltpu.make_async_remote_copy(
                src_ref=comm.at[li, 0], dst_ref=comm.at[li, 2],
                send_sem=send_sems.at[s + 1], recv_sem=recv_sems.at[s + 1],
                device_id=(left,), device_id_type=pl.DeviceIdType.MESH)
            r_d = pltpu.make_async_remote_copy(
                src_ref=comm.at[li, 0], dst_ref=comm.at[li, 3],
                send_sem=send_sems.at[s + 2], recv_sem=recv_sems.at[s + 2],
                device_id=(diag,), device_id_type=pl.DeviceIdType.MESH)
            r_r.start()
            r_l.start()
            r_d.start()

            if li + 1 < N_LAYERS:
                cast_win(li + 1)
                if li + 2 < N_LAYERS:
                    dma_in(wins_hbm[li + 2], sin, li % 2, 2 * (li + 2)).start()
                cast_wout(li + 1)
                if li + 2 < N_LAYERS:
                    dma_in(wouts_hbm[li + 2], sout, li % 2,
                           2 * (li + 2) + 1).start()

            r_r.wait_recv()
            r_l.wait_recv()
            acc = (partial
                   + comm[li, 1].astype(jnp.float32)
                   + comm[li, 2].astype(jnp.float32))
            r_d.wait_recv()
            x_val = acc + comm[li, 3].astype(jnp.float32)

            r_r.wait_send()
            r_l.wait_send()
            r_d.wait_send()

        out_ref[...] = x_val

    return pl.pallas_call(
        body,
        out_shape=jax.ShapeDtypeStruct((b, d), jnp.float32),
        in_specs=[pl.BlockSpec(memory_space=pltpu.VMEM)]
        + [pl.BlockSpec(memory_space=pltpu.MemorySpace.HBM)] * 6,
        out_specs=pl.BlockSpec(memory_space=pltpu.VMEM),
        scratch_shapes=[
            pltpu.VMEM((2, d, hsh), jnp.float32),
            pltpu.VMEM((2, hsh, d), jnp.float32),
            pltpu.VMEM((2, d, hsh), jnp.bfloat16),
            pltpu.VMEM((2, hsh, d), jnp.bfloat16),
            pltpu.VMEM((N_LAYERS, 4, b, d), jnp.bfloat16),
            pltpu.SemaphoreType.DMA((2 * N_LAYERS,)),
            pltpu.SemaphoreType.DMA((3 * N_LAYERS,)),
            pltpu.SemaphoreType.DMA((3 * N_LAYERS,)),
        ],
        compiler_params=pltpu.CompilerParams(
            collective_id=0,
            vmem_limit_bytes=100 * 1024 * 1024,
        ),
    )(x, Win0, Wout0, Win1, Wout1, Win2, Wout2)
